# Initial kernel scaffold; baseline (speedup 1.0000x reference)
import jax
import jax.numpy as jnp
from jax.experimental import pallas as pl


def kernel(x, W, b):
    raise NotImplementedError("write your pallas kernel here")


# speedup vs baseline: 20.4088x; 20.4088x over previous
import jax
import jax.numpy as jnp
from jax.experimental import pallas as pl
from jax.experimental.pallas import tpu as pltpu

B, H, S = 1, 12, 2048
_ROWS = 256


def _phase1_kernel(xs_ref, w_ref, b_ref, g_ref):
    m = jnp.max(xs_ref[0], axis=1)
    a = jax.lax.dot_general(
        m[None, :], w_ref[...],
        dimension_numbers=(((1,), (1,)), ((), ())),
        preferred_element_type=jnp.float32,
    )
    a = a + b_ref[...]
    g_ref[...] = 0.5 * a * (1.0 + jax.lax.erf(a * 0.7071067811865476))


def _phase2_kernel(x_ref, g_ref, o_ref):
    blk = x_ref[0]
    m = jnp.max(blk, axis=-1, keepdims=True)
    lane = jax.lax.broadcasted_iota(jnp.int32, blk.shape, 1)
    cand = jnp.where(blk == m, lane, S)
    idx = jnp.min(cand, axis=-1, keepdims=True)
    o_ref[0] = jnp.where(lane == idx, g_ref[0][:, None], blk)


def kernel(x, W, b):
    x3 = x.reshape(H, S, S)
    g = pl.pallas_call(
        _phase1_kernel,
        grid=(1,),
        in_specs=[
            pl.BlockSpec((1, S, S), lambda i: (H - 1, 0, 0)),
            pl.BlockSpec((S, S), lambda i: (0, 0)),
            pl.BlockSpec((1, S), lambda i: (0, 0)),
        ],
        out_specs=pl.BlockSpec((1, S), lambda i: (0, 0)),
        out_shape=jax.ShapeDtypeStruct((1, S), jnp.float32),
    )(x3, W, b.reshape(1, S))

    out = pl.pallas_call(
        _phase2_kernel,
        grid=(H, S // _ROWS),
        in_specs=[
            pl.BlockSpec((1, _ROWS, S), lambda h, i: (h, i, 0)),
            pl.BlockSpec((1, _ROWS), lambda h, i: (0, i)),
        ],
        out_specs=pl.BlockSpec((1, _ROWS, S), lambda h, i: (h, i, 0)),
        out_shape=jax.ShapeDtypeStruct((H, S, S), jnp.float32),
        compiler_params=pltpu.CompilerParams(
            dimension_semantics=("parallel", "parallel"),
        ),
    )(x3, g)
    return out.reshape(B, H, S, S)
